# baseline (speedup 1.0000x reference)
import functools

import jax
import jax.numpy as jnp
from jax.experimental import pallas as pl
from jax.experimental.pallas import tpu as pltpu

_ROTS = ((13, 15, 26, 6), (17, 29, 16, 24))


def _gumbel_bits(lin):
    k0 = jnp.uint32(0)
    k1 = jnp.uint32(42)
    k2 = k0 ^ k1 ^ jnp.uint32(0x1BD11BDA)
    ks = (k0, k1, k2)
    x0 = jnp.zeros_like(lin)
    x1 = lin + k1
    for i in range(5):
        for r in _ROTS[i % 2]:
            x0 = x0 + x1
            x1 = (x1 << jnp.uint32(r)) | (x1 >> jnp.uint32(32 - r))
            x1 = x0 ^ x1
        x0 = x0 + ks[(i + 1) % 3]
        x1 = x1 + ks[(i + 2) % 3] + jnp.uint32(i + 1)
    return x0 ^ x1


def _body(x_ref, samples_ref, lp_ref, acc_m, acc_i, acc_x, acc_s, *, cb, v):
    i = pl.program_id(0)
    g = pl.num_programs(0)

    @pl.when(i == 0)
    def _init():
        acc_m[...] = jnp.full_like(acc_m, -jnp.inf)
        acc_i[...] = jnp.zeros_like(acc_i)
        acc_x[...] = jnp.zeros_like(acc_x)
        acc_s[...] = jnp.zeros_like(acc_s)

    x = x_ref[...]
    b, _ = x.shape
    col = jax.lax.broadcasted_iota(jnp.int32, (b, cb), 1) + i * cb
    roff = jax.lax.broadcasted_iota(jnp.int32, (b, 1), 0) * v
    lin = (roff + col).astype(jnp.uint32)

    bits = _gumbel_bits(lin)
    u = jax.lax.bitcast_convert_type(
        (bits >> jnp.uint32(9)) | jnp.uint32(0x3F800000), jnp.float32) - 1.0
    gum = -jnp.log(-jnp.log(u))

    valid = col < v
    y = jnp.where(valid, x + gum, -jnp.inf)

    bm = jnp.max(y, axis=1, keepdims=True)
    at_max = y == bm
    bidx = jnp.min(jnp.where(at_max, col, jnp.int32(0x7FFFFFFF)),
                   axis=1, keepdims=True)
    bx = jnp.sum(jnp.where(col == bidx, x, 0.0), axis=1, keepdims=True)
    bs = jnp.sum(jnp.where(valid, jnp.exp(x), 0.0), axis=1, keepdims=True)

    upd = bm > acc_m[...]
    acc_i[...] = jnp.where(upd, bidx, acc_i[...])
    acc_x[...] = jnp.where(upd, bx, acc_x[...])
    acc_m[...] = jnp.where(upd, bm, acc_m[...])
    acc_s[...] = acc_s[...] + bs

    @pl.when(i == g - 1)
    def _finish():
        samples_ref[...] = acc_i[...]
        lp_ref[...] = acc_x[...] - jnp.log(acc_s[...])


def kernel(logits):
    b, v = logits.shape
    cb = 16384
    grid = (pl.cdiv(v, cb),)
    samples, lp = pl.pallas_call(
        functools.partial(_body, cb=cb, v=v),
        grid=grid,
        in_specs=[pl.BlockSpec((b, cb), lambda i: (0, i))],
        out_specs=[pl.BlockSpec((b, 1), lambda i: (0, 0)),
                   pl.BlockSpec((b, 1), lambda i: (0, 0))],
        out_shape=[jax.ShapeDtypeStruct((b, 1), jnp.int32),
                   jax.ShapeDtypeStruct((b, 1), jnp.float32)],
        scratch_shapes=[
            pltpu.VMEM((b, 1), jnp.float32),
            pltpu.VMEM((b, 1), jnp.int32),
            pltpu.VMEM((b, 1), jnp.float32),
            pltpu.VMEM((b, 1), jnp.float32),
        ],
    )(logits)
    return samples[:, 0], lp[:, 0]


# speedup vs baseline: 1.3567x; 1.3567x over previous
import functools

import jax
import jax.numpy as jnp
from jax.experimental import pallas as pl
from jax.experimental.pallas import tpu as pltpu

_ROTS = ((13, 15, 26, 6), (17, 29, 16, 24))


def _gumbel_bits(lin):
    k0 = jnp.uint32(0)
    k1 = jnp.uint32(42)
    k2 = k0 ^ k1 ^ jnp.uint32(0x1BD11BDA)
    ks = (k0, k1, k2)
    x0 = jnp.zeros_like(lin)
    x1 = lin + k1
    for i in range(5):
        for r in _ROTS[i % 2]:
            x0 = x0 + x1
            x1 = (x1 << jnp.uint32(r)) | (x1 >> jnp.uint32(32 - r))
            x1 = x0 ^ x1
        x0 = x0 + ks[(i + 1) % 3]
        x1 = x1 + ks[(i + 2) % 3] + jnp.uint32(i + 1)
    return x0 ^ x1


def _body(x_ref, samples_ref, lp_ref, acc_m, acc_i, acc_x, acc_s, *, cb, v):
    i = pl.program_id(0)
    g = pl.num_programs(0)

    @pl.when(i == 0)
    def _init():
        acc_m[...] = jnp.full_like(acc_m, -jnp.inf)
        acc_i[...] = jnp.zeros_like(acc_i)
        acc_x[...] = jnp.zeros_like(acc_x)
        acc_s[...] = jnp.zeros_like(acc_s)

    x = x_ref[...]
    b, _ = x.shape
    col = jax.lax.broadcasted_iota(jnp.int32, (b, cb), 1) + i * cb
    roff = jax.lax.broadcasted_iota(jnp.int32, (b, 1), 0) * v
    lin = (roff + col).astype(jnp.uint32)

    bits = _gumbel_bits(lin)
    u = jax.lax.bitcast_convert_type(
        (bits >> jnp.uint32(9)) | jnp.uint32(0x3F800000), jnp.float32) - 1.0
    gum = -jnp.log(-jnp.log(u))

    valid = col < v
    y = jnp.where(valid, x + gum, -jnp.inf)

    bm = jnp.max(y, axis=1, keepdims=True)
    at_max = y == bm
    bidx = jnp.min(jnp.where(at_max, col, jnp.int32(0x7FFFFFFF)),
                   axis=1, keepdims=True)
    bx = jnp.sum(jnp.where(col == bidx, x, 0.0), axis=1, keepdims=True)
    bs = jnp.sum(jnp.where(valid, jnp.exp(x), 0.0), axis=1, keepdims=True)

    upd = bm > acc_m[...]
    acc_i[...] = jnp.where(upd, bidx, acc_i[...])
    acc_x[...] = jnp.where(upd, bx, acc_x[...])
    acc_m[...] = jnp.where(upd, bm, acc_m[...])
    acc_s[...] = acc_s[...] + bs

    @pl.when(i == g - 1)
    def _finish():
        samples_ref[...] = acc_i[...]
        lp_ref[...] = acc_x[...] - jnp.log(acc_s[...])


def kernel(logits):
    b, v = logits.shape
    cb = 4096
    grid = (pl.cdiv(v, cb),)
    samples, lp = pl.pallas_call(
        functools.partial(_body, cb=cb, v=v),
        grid=grid,
        in_specs=[pl.BlockSpec((b, cb), lambda i: (0, i))],
        out_specs=[pl.BlockSpec((b, 1), lambda i: (0, 0)),
                   pl.BlockSpec((b, 1), lambda i: (0, 0))],
        out_shape=[jax.ShapeDtypeStruct((b, 1), jnp.int32),
                   jax.ShapeDtypeStruct((b, 1), jnp.float32)],
        scratch_shapes=[
            pltpu.VMEM((b, 1), jnp.float32),
            pltpu.VMEM((b, 1), jnp.int32),
            pltpu.VMEM((b, 1), jnp.float32),
            pltpu.VMEM((b, 1), jnp.float32),
        ],
    )(logits)
    return samples[:, 0], lp[:, 0]


# speedup vs baseline: 1.4510x; 1.0695x over previous
import functools

import jax
import jax.numpy as jnp
from jax.experimental import pallas as pl
from jax.experimental.pallas import tpu as pltpu

_ROTS = ((13, 15, 26, 6), (17, 29, 16, 24))


def _gumbel_bits(lin):
    k0 = jnp.uint32(0)
    k1 = jnp.uint32(42)
    k2 = k0 ^ k1 ^ jnp.uint32(0x1BD11BDA)
    ks = (k0, k1, k2)
    x0 = jnp.zeros_like(lin)
    x1 = lin + k1
    for i in range(5):
        for r in _ROTS[i % 2]:
            x0 = x0 + x1
            x1 = (x1 << jnp.uint32(r)) | (x1 >> jnp.uint32(32 - r))
            x1 = x0 ^ x1
        x0 = x0 + ks[(i + 1) % 3]
        x1 = x1 + ks[(i + 2) % 3] + jnp.uint32(i + 1)
    return x0 ^ x1


def _body(x_ref, samples_ref, lp_ref, acc_m, acc_i, acc_x, acc_s, *, cb, v):
    i = pl.program_id(0)
    g = pl.num_programs(0)

    @pl.when(i == 0)
    def _init():
        acc_m[...] = jnp.full_like(acc_m, -jnp.inf)
        acc_i[...] = jnp.zeros_like(acc_i)
        acc_x[...] = jnp.zeros_like(acc_x)
        acc_s[...] = jnp.zeros_like(acc_s)

    x = x_ref[...]
    b, _ = x.shape
    col = jax.lax.broadcasted_iota(jnp.int32, (b, cb), 1) + i * cb
    roff = jax.lax.broadcasted_iota(jnp.int32, (b, 1), 0) * v
    lin = (roff + col).astype(jnp.uint32)

    bits = _gumbel_bits(lin)
    u = jax.lax.bitcast_convert_type(
        (bits >> jnp.uint32(9)) | jnp.uint32(0x3F800000), jnp.float32) - 1.0
    gum = -jnp.log(-jnp.log(u))

    valid = col < v
    y = jnp.where(valid, x + gum, -jnp.inf)

    bm = jnp.max(y, axis=1, keepdims=True)
    at_max = y == bm
    bidx = jnp.min(jnp.where(at_max, col, jnp.int32(0x7FFFFFFF)),
                   axis=1, keepdims=True)
    bx = jnp.sum(jnp.where(col == bidx, x, 0.0), axis=1, keepdims=True)
    bs = jnp.sum(jnp.where(valid, jnp.exp(x), 0.0), axis=1, keepdims=True)

    upd = bm > acc_m[...]
    acc_i[...] = jnp.where(upd, bidx, acc_i[...])
    acc_x[...] = jnp.where(upd, bx, acc_x[...])
    acc_m[...] = jnp.where(upd, bm, acc_m[...])
    acc_s[...] = acc_s[...] + bs

    @pl.when(i == g - 1)
    def _finish():
        samples_ref[...] = acc_i[...]
        lp_ref[...] = acc_x[...] - jnp.log(acc_s[...])


def kernel(logits):
    b, v = logits.shape
    cb = 2048
    grid = (pl.cdiv(v, cb),)
    samples, lp = pl.pallas_call(
        functools.partial(_body, cb=cb, v=v),
        grid=grid,
        in_specs=[pl.BlockSpec((b, cb), lambda i: (0, i))],
        out_specs=[pl.BlockSpec((b, 1), lambda i: (0, 0)),
                   pl.BlockSpec((b, 1), lambda i: (0, 0))],
        out_shape=[jax.ShapeDtypeStruct((b, 1), jnp.int32),
                   jax.ShapeDtypeStruct((b, 1), jnp.float32)],
        scratch_shapes=[
            pltpu.VMEM((b, 1), jnp.float32),
            pltpu.VMEM((b, 1), jnp.int32),
            pltpu.VMEM((b, 1), jnp.float32),
            pltpu.VMEM((b, 1), jnp.float32),
        ],
    )(logits)
    return samples[:, 0], lp[:, 0]
